# baseline (speedup 1.0000x reference)
import functools

import jax
import jax.numpy as jnp
from jax import lax
from jax.experimental import pallas as pl

NUM_TOKENS = 32768
D_MODEL = 4096
NUM_EXPERTS = 64

TM = 512


def _router_kernel(x_ref, wt_ref, oh_ref, top_ref, logits_ref):
    logits = jnp.dot(x_ref[...], wt_ref[...], preferred_element_type=jnp.float32)
    m = jnp.max(logits, axis=1, keepdims=True)
    s = jnp.sum(jnp.exp(logits - m), axis=1, keepdims=True)
    ii = lax.broadcasted_iota(jnp.int32, logits.shape, 1)
    cand = jnp.where(logits == m, ii, NUM_EXPERTS)
    first = jnp.min(cand, axis=1, keepdims=True)
    oh_ref[...] = (ii == first).astype(jnp.int32)
    top_ref[...] = 1.0 / s
    logits_ref[...] = logits


@jax.jit
def kernel(x, W):
    wt = W.T
    grid = (NUM_TOKENS // TM,)
    oh, top, logits = pl.pallas_call(
        _router_kernel,
        grid=grid,
        in_specs=[
            pl.BlockSpec((TM, D_MODEL), lambda i: (i, 0)),
            pl.BlockSpec((D_MODEL, NUM_EXPERTS), lambda i: (0, 0)),
        ],
        out_specs=[
            pl.BlockSpec((TM, NUM_EXPERTS), lambda i: (i, 0)),
            pl.BlockSpec((TM, 1), lambda i: (i, 0)),
            pl.BlockSpec((TM, NUM_EXPERTS), lambda i: (i, 0)),
        ],
        out_shape=[
            jax.ShapeDtypeStruct((NUM_TOKENS, NUM_EXPERTS), jnp.int32),
            jax.ShapeDtypeStruct((NUM_TOKENS, 1), jnp.float32),
            jax.ShapeDtypeStruct((NUM_TOKENS, NUM_EXPERTS), jnp.float32),
        ],
    )(x, wt)
    return oh, top, logits


# speedup vs baseline: 1.0196x; 1.0196x over previous
import functools

import jax
import jax.numpy as jnp
from jax import lax
from jax.experimental import pallas as pl

NUM_TOKENS = 32768
D_MODEL = 4096
NUM_EXPERTS = 64

TM = 1024


def _router_kernel(x_ref, wt_ref, oh_ref, top_ref, logits_ref):
    logits = jnp.dot(x_ref[...], wt_ref[...], preferred_element_type=jnp.float32)
    m = jnp.max(logits, axis=1, keepdims=True)
    s = jnp.sum(jnp.exp(logits - m), axis=1, keepdims=True)
    ii = lax.broadcasted_iota(jnp.int32, logits.shape, 1)
    cand = jnp.where(logits == m, ii, NUM_EXPERTS)
    first = jnp.min(cand, axis=1, keepdims=True)
    oh_ref[...] = (ii == first).astype(jnp.int32)
    top_ref[...] = 1.0 / s
    logits_ref[...] = logits


@jax.jit
def kernel(x, W):
    wt = W.T
    grid = (NUM_TOKENS // TM,)
    oh, top, logits = pl.pallas_call(
        _router_kernel,
        grid=grid,
        in_specs=[
            pl.BlockSpec((TM, D_MODEL), lambda i: (i, 0)),
            pl.BlockSpec((D_MODEL, NUM_EXPERTS), lambda i: (0, 0)),
        ],
        out_specs=[
            pl.BlockSpec((TM, NUM_EXPERTS), lambda i: (i, 0)),
            pl.BlockSpec((TM, 1), lambda i: (i, 0)),
            pl.BlockSpec((TM, NUM_EXPERTS), lambda i: (i, 0)),
        ],
        out_shape=[
            jax.ShapeDtypeStruct((NUM_TOKENS, NUM_EXPERTS), jnp.int32),
            jax.ShapeDtypeStruct((NUM_TOKENS, 1), jnp.float32),
            jax.ShapeDtypeStruct((NUM_TOKENS, NUM_EXPERTS), jnp.float32),
        ],
    )(x, wt)
    return oh, top, logits
